# baseline (speedup 1.0000x reference)
import functools

import jax
import jax.numpy as jnp
import numpy as np
from jax import lax
from jax.experimental import pallas as pl
from jax.experimental.pallas import tpu as pltpu
from jax.experimental.pallas import tpu_sc as plsc

N = 10000
E = 320000
C = 128
H = 8
DH = C // H
R = 16
FFN = 4 * C
EPS = 1.1920928955078125e-07
BN = 1000
BNP = 1024

NW = 32
EP = 327680
EPAD = EP + 64
EWP = EP // NW
CH = 32
NCHUNK = EWP // CH
NG = CH // 16
NP = 10240
NROW = NP // 16


def _qkv_body(x_ref, n1_ref, wq_ref, bq_ref, wk_ref, bk_ref, wv_ref, bv_ref,
              q_ref, kv_ref):
    x = x_ref[...]
    xn = x * lax.rsqrt(jnp.mean(x * x, axis=-1, keepdims=True) + EPS)
    xn = xn * n1_ref[...]
    q_ref[...] = jnp.dot(xn, wq_ref[...], preferred_element_type=jnp.float32) + bq_ref[...]
    k = jnp.dot(xn, wk_ref[...], preferred_element_type=jnp.float32) + bk_ref[...]
    v = jnp.dot(xn, wv_ref[...], preferred_element_type=jnp.float32) + bv_ref[...]
    kv_ref[...] = jnp.concatenate([k, v], axis=1)


def _qkv(xp, n1, wq, bq, wk, bk, wv, bv):
    row = pl.BlockSpec((BNP, C), lambda i: (i, 0))
    row2 = pl.BlockSpec((BNP, 2 * C), lambda i: (i, 0))
    full2 = pl.BlockSpec((C, C), lambda i: (0, 0))
    vec = pl.BlockSpec((1, C), lambda i: (0, 0))
    return pl.pallas_call(
        _qkv_body,
        grid=(NP // BNP,),
        in_specs=[row, vec, full2, vec, full2, vec, full2, vec],
        out_specs=[row, row2],
        out_shape=[jax.ShapeDtypeStruct((NP, C), jnp.float32),
                   jax.ShapeDtypeStruct((NP, 2 * C), jnp.float32)],
    )(xp, n1.reshape(1, C), wq, bq.reshape(1, C), wk, bk.reshape(1, C),
      wv, bv.reshape(1, C))


def _tail_body(x_ref, n0_ref, n1_ref, d0_ref, d1_ref, exp_ref,
               wo_ref, bo_ref, norm2_ref, w1_ref, b1_ref, w2_ref, b2_ref,
               o_ref):
    numer = n0_ref[...] + n1_ref[...]
    den = d0_ref[...] + d1_ref[...]
    rec = 1.0 / (den[:, :H] + 1e-16)
    rece = jnp.dot(rec, exp_ref[...], preferred_element_type=jnp.float32)
    attn = numer * rece
    x = x_ref[...]
    y = x + jnp.dot(attn, wo_ref[...], preferred_element_type=jnp.float32) + bo_ref[...]
    xn = y * lax.rsqrt(jnp.mean(y * y, axis=-1, keepdims=True) + EPS)
    xn = xn * norm2_ref[...]
    h = jnp.dot(xn, w1_ref[...], preferred_element_type=jnp.float32) + b1_ref[...]
    h = 0.5 * h * (1.0 + lax.erf(h * np.float32(1.0 / np.sqrt(2.0))))
    o_ref[...] = y + jnp.dot(h, w2_ref[...], preferred_element_type=jnp.float32) + b2_ref[...]


def _tail(x, n0, n1, d0, d1, wo, bo, n2, w1, b1, w2, b2):
    row = pl.BlockSpec((BN, C), lambda i: (i, 0))
    drow = pl.BlockSpec((BN, 16), lambda i: (i, 0))
    vec = pl.BlockSpec((1, C), lambda i: (0, 0))
    expand = jnp.asarray(np.kron(np.eye(H), np.ones((1, DH))), dtype=jnp.float32)
    return pl.pallas_call(
        _tail_body,
        grid=(N // BN,),
        in_specs=[row, row, row, drow, drow,
                  pl.BlockSpec((H, C), lambda i: (0, 0)),
                  pl.BlockSpec((C, C), lambda i: (0, 0)), vec, vec,
                  pl.BlockSpec((C, FFN), lambda i: (0, 0)),
                  pl.BlockSpec((1, FFN), lambda i: (0, 0)),
                  pl.BlockSpec((FFN, C), lambda i: (0, 0)), vec],
        out_specs=row,
        out_shape=jax.ShapeDtypeStruct((N, C), jnp.float32),
    )(x, n0, n1, d0, d1, expand, wo, bo.reshape(1, C), n2.reshape(1, C),
      w1, b1.reshape(1, FFN), w2, b2.reshape(1, C))


def _in_proj_body(x_ref, w_ref, b_ref, o_ref):
    o_ref[...] = jnp.dot(x_ref[...], w_ref[...], preferred_element_type=jnp.float32) + b_ref[...]


def _in_proj(x, w, b):
    row = pl.BlockSpec((BN, C), lambda i: (i, 0))
    return pl.pallas_call(
        _in_proj_body,
        grid=(N // BN,),
        in_specs=[row, pl.BlockSpec((C, C), lambda i: (0, 0)),
                  pl.BlockSpec((1, C), lambda i: (0, 0))],
        out_specs=row,
        out_shape=jax.ShapeDtypeStruct((N, C), jnp.float32),
    )(x, w, b.reshape(1, C))


def _dyn_gather(vec, idx):
    return lax.gather(vec, idx[:, None],
                      dimension_numbers=lax.GatherDimensionNumbers(
                          offset_dims=(), collapsed_slice_dims=(0,),
                          start_index_map=(0,)),
                      slice_sizes=(1,),
                      mode=lax.GatherScatterMode.PROMISE_IN_BOUNDS)


def _edge_body(q_hbm, kv_hbm, rel_hbm, src_hbm, dst_hbm, et_hbm,
               zn_hbm, zd_hbm,
               onum, oden,
               qb0, qb1, kvb0, kvb1, wb0, wb1, xb0, xb1, relbuf,
               sb0, sb1, db0, db1, eb0, eb1, sdb0, sdb1,
               num_sh, den_sh, isem, gsem, ssem0, ssem1):
    qb = (qb0, qb1)
    kvb = (kvb0, kvb1)
    wb = (wb0, wb1)
    xb = (xb0, xb1)
    sb = (sb0, sb1)
    db = (db0, db1)
    eb = (eb0, eb1)
    sdb = (sdb0, sdb1)
    ssem = (ssem0, ssem1)

    cid = lax.axis_index("c")
    sid = lax.axis_index("s")
    wid = sid * 2 + cid
    pltpu.sync_copy(rel_hbm, relbuf)
    pltpu.sync_copy(zn_hbm, wb0)
    pltpu.sync_copy(zd_hbm, xb0)
    row0 = sid * NROW
    for t in range(NROW // CH):
        pltpu.sync_copy(wb0, num_sh.at[pl.ds(row0 + t * CH, CH)])
        pltpu.sync_copy(xb0, den_sh.at[pl.ds(row0 + t * CH, CH)])
    plsc.subcore_barrier()

    ebase = wid * EWP
    iota16 = lax.iota(jnp.int32, 16)

    c1 = pltpu.async_copy(src_hbm.at[pl.ds(ebase, CH)], sb0, isem)
    c2 = pltpu.async_copy(dst_hbm.at[pl.ds(ebase, CH)], db0, isem)
    c3 = pltpu.async_copy(et_hbm.at[pl.ds(ebase, CH)], eb0, isem)
    c1.wait()
    c2.wait()
    c3.wait()
    pltpu.async_copy(q_hbm.at[db0], qb0, gsem)
    pltpu.async_copy(kv_hbm.at[sb0], kvb0, gsem)
    pltpu.async_copy(src_hbm.at[pl.ds(ebase + CH, CH)], sb1, isem)
    pltpu.async_copy(dst_hbm.at[pl.ds(ebase + CH, CH)], db1, isem)
    pltpu.async_copy(et_hbm.at[pl.ds(ebase + CH, CH)], eb1, isem)

    def step(i2, carry):
        for b in (0, 1):
            ob = 1 - b
            cbase = ebase + (i2 * 2 + b) * CH
            pltpu.make_async_copy(q_hbm.at[pl.ds(0, CH)], qb[b], gsem).wait()
            pltpu.make_async_copy(kv_hbm.at[pl.ds(0, CH)], kvb[b], gsem).wait()
            pltpu.make_async_copy(src_hbm.at[pl.ds(0, CH)], sb[ob], isem).wait()
            pltpu.make_async_copy(dst_hbm.at[pl.ds(0, CH)], db[ob], isem).wait()
            pltpu.make_async_copy(et_hbm.at[pl.ds(0, CH)], eb[ob], isem).wait()
            pltpu.async_copy(q_hbm.at[db[ob]], qb[ob], gsem)
            pltpu.async_copy(kv_hbm.at[sb[ob]], kvb[ob], gsem)


            def group(g, c2):
                et16 = eb[b][pl.ds(g * 16, 16)]
                for e in range(16):
                    erow = g * 16 + e
                    ete = _dyn_gather(et16, jnp.full((16,), e, jnp.int32))
                    rbase = ete * C + iota16
                    qv = [qb[b][erow, pl.ds(h * DH, 16)] for h in range(H)]
                    kv = [kvb[b][erow, pl.ds(h * DH, 16)] for h in range(H)]
                    vv = [kvb[b][erow, pl.ds(C + h * DH, 16)] for h in range(H)]
                    rv = [plsc.load_gather(relbuf, [rbase + h * DH])
                          for h in range(H)]
                    p = [qv[h] * (kv[h] + rv[h]) for h in range(H)]
                    for sh in (8, 4, 2, 1):
                        pg = [_dyn_gather(p[h], iota16 ^ sh) for h in range(H)]
                        p = [p[h] + pg[h] for h in range(H)]
                    ex = [jnp.exp(p[h] * np.float32(1.0 / np.sqrt(DH)))
                          for h in range(H)]
                    w = [(vv[h] + rv[h]) * ex[h] for h in range(H)]
                    for h in range(H):
                        wb[b][erow, pl.ds(h * DH, 16)] = w[h]
                    exrow = jnp.zeros((16,), jnp.float32)
                    for h in range(H):
                        exrow = jnp.where(iota16 == h, ex[h], exrow)
                    xb[b][erow, pl.ds(0, 16)] = exrow
                return c2
            lax.fori_loop(0, NG, group, 0)

            sdb[b][pl.ds(0, 16)] = db[b][pl.ds(0, 16)]
            sdb[b][pl.ds(16, 16)] = db[b][pl.ds(16, 16)]
            pltpu.async_copy(src_hbm.at[pl.ds(cbase + 2 * CH, CH)], sb[b], isem)
            pltpu.async_copy(dst_hbm.at[pl.ds(cbase + 2 * CH, CH)], db[b], isem)
            pltpu.async_copy(et_hbm.at[pl.ds(cbase + 2 * CH, CH)], eb[b], isem)
        return carry
    lax.fori_loop(0, NCHUNK // 2, step, 0)

    pltpu.make_async_copy(q_hbm.at[pl.ds(0, CH)], qb0, gsem).wait()
    pltpu.make_async_copy(kv_hbm.at[pl.ds(0, CH)], kvb0, gsem).wait()
    pltpu.make_async_copy(src_hbm.at[pl.ds(0, CH)], sb1, isem).wait()
    pltpu.make_async_copy(dst_hbm.at[pl.ds(0, CH)], db1, isem).wait()
    pltpu.make_async_copy(et_hbm.at[pl.ds(0, CH)], eb1, isem).wait()
    plsc.subcore_barrier()
    out_base = cid * NP + row0
    for t in range(NROW // CH):
        pltpu.sync_copy(num_sh.at[pl.ds(row0 + t * CH, CH)], wb0)
        pltpu.sync_copy(wb0, onum.at[pl.ds(out_base + t * CH, CH)])
        pltpu.sync_copy(den_sh.at[pl.ds(row0 + t * CH, CH)], xb0)
        pltpu.sync_copy(xb0, oden.at[pl.ds(out_base + t * CH, CH)])


def _edge_phase(q, kv, rel, srcp, dstp, etp):
    mesh = plsc.VectorSubcoreMesh(core_axis_name="c", subcore_axis_name="s")
    fn = pl.kernel(
        _edge_body,
        mesh=mesh,
        compiler_params=pltpu.CompilerParams(needs_layout_passes=False,
                                             use_tc_tiling_on_sc=False),
        out_type=[jax.ShapeDtypeStruct((2 * NP, C), jnp.float32),
                  jax.ShapeDtypeStruct((2 * NP, 16), jnp.float32)],
        scratch_types=[
            pltpu.VMEM((CH, C), jnp.float32),
            pltpu.VMEM((CH, C), jnp.float32),
            pltpu.VMEM((CH, 2 * C), jnp.float32),
            pltpu.VMEM((CH, 2 * C), jnp.float32),
            pltpu.VMEM((CH, C), jnp.float32),
            pltpu.VMEM((CH, C), jnp.float32),
            pltpu.VMEM((CH, 16), jnp.float32),
            pltpu.VMEM((CH, 16), jnp.float32),
            pltpu.VMEM((R * C,), jnp.float32),
            pltpu.VMEM((CH,), jnp.int32),
            pltpu.VMEM((CH,), jnp.int32),
            pltpu.VMEM((CH,), jnp.int32),
            pltpu.VMEM((CH,), jnp.int32),
            pltpu.VMEM((CH,), jnp.int32),
            pltpu.VMEM((CH,), jnp.int32),
            pltpu.VMEM((CH,), jnp.int32),
            pltpu.VMEM((CH,), jnp.int32),
            pltpu.VMEM_SHARED((NP, C), jnp.float32),
            pltpu.VMEM_SHARED((NP, 16), jnp.float32),
            pltpu.SemaphoreType.DMA,
            pltpu.SemaphoreType.DMA,
            pltpu.SemaphoreType.DMA,
            pltpu.SemaphoreType.DMA,
        ],
    )
    zn = jnp.zeros((CH, C), jnp.float32)
    zd = jnp.zeros((CH, 16), jnp.float32)
    onum, oden = fn(q, kv, rel.reshape(-1), srcp, dstp, etp, zn, zd)
    return onum, oden


def kernel(x, edge_index, edge_type, params):
    pad = EPAD - E
    srcp = jnp.concatenate([edge_index[0], jnp.zeros((pad,), edge_index.dtype)])
    dstp = jnp.concatenate([edge_index[1], jnp.full((pad,), N, edge_index.dtype)])
    etp = jnp.concatenate([edge_type, jnp.zeros((pad,), edge_type.dtype)])
    xpad = jnp.zeros((NP - N, C), jnp.float32)
    p0 = params["input_proj"]
    x = _in_proj(x, p0["w"], p0["b"])
    for p in params["blocks"]:
        xp = jnp.concatenate([x, xpad])
        q, kv = _qkv(xp, p["norm1"], p["q"]["w"], p["q"]["b"],
                     p["k"]["w"], p["k"]["b"], p["v"]["w"], p["v"]["b"])
        onum, oden = _edge_phase(q, kv, p["rel"], srcp, dstp, etp)
        x = _tail(x, onum[:N], onum[NP:NP + N], oden[:N], oden[NP:NP + N],
                  p["out"]["w"], p["out"]["b"], p["norm2"],
                  p["ffn1"]["w"], p["ffn1"]["b"], p["ffn2"]["w"], p["ffn2"]["b"])
    return x


# speedup vs baseline: 1.2618x; 1.2618x over previous
import functools

import jax
import jax.numpy as jnp
import numpy as np
from jax import lax
from jax.experimental import pallas as pl
from jax.experimental.pallas import tpu as pltpu
from jax.experimental.pallas import tpu_sc as plsc

N = 10000
E = 320000
C = 128
H = 8
DH = C // H
R = 16
FFN = 4 * C
EPS = 1.1920928955078125e-07
BN = 1000
BNP = 1024

NW = 32
EP = 327680
EPAD = EP + 64
EWP = EP // NW
CH = 32
NCHUNK = EWP // CH
NG = CH // 16
NP = 10240
NROW = NP // 16


def _qkv_body(x_ref, n1_ref, wq_ref, bq_ref, wk_ref, bk_ref, wv_ref, bv_ref,
              q_ref, kv_ref):
    x = x_ref[...]
    xn = x * lax.rsqrt(jnp.mean(x * x, axis=-1, keepdims=True) + EPS)
    xn = xn * n1_ref[...]
    q_ref[...] = jnp.dot(xn, wq_ref[...], preferred_element_type=jnp.float32) + bq_ref[...]
    k = jnp.dot(xn, wk_ref[...], preferred_element_type=jnp.float32) + bk_ref[...]
    v = jnp.dot(xn, wv_ref[...], preferred_element_type=jnp.float32) + bv_ref[...]
    kv_ref[...] = jnp.concatenate([k, v], axis=1)


def _qkv(xp, n1, wq, bq, wk, bk, wv, bv):
    row = pl.BlockSpec((BNP, C), lambda i: (i, 0))
    row2 = pl.BlockSpec((BNP, 2 * C), lambda i: (i, 0))
    full2 = pl.BlockSpec((C, C), lambda i: (0, 0))
    vec = pl.BlockSpec((1, C), lambda i: (0, 0))
    return pl.pallas_call(
        _qkv_body,
        grid=(NP // BNP,),
        in_specs=[row, vec, full2, vec, full2, vec, full2, vec],
        out_specs=[row, row2],
        out_shape=[jax.ShapeDtypeStruct((NP, C), jnp.float32),
                   jax.ShapeDtypeStruct((NP, 2 * C), jnp.float32)],
    )(xp, n1.reshape(1, C), wq, bq.reshape(1, C), wk, bk.reshape(1, C),
      wv, bv.reshape(1, C))


def _tail_body(x_ref, n0_ref, n1_ref, d0_ref, d1_ref, exp_ref,
               wo_ref, bo_ref, norm2_ref, w1_ref, b1_ref, w2_ref, b2_ref,
               o_ref):
    numer = n0_ref[...] + n1_ref[...]
    den = d0_ref[...] + d1_ref[...]
    rec = 1.0 / (den[:, :H] + 1e-16)
    rece = jnp.dot(rec, exp_ref[...], preferred_element_type=jnp.float32)
    attn = numer * rece
    x = x_ref[...]
    y = x + jnp.dot(attn, wo_ref[...], preferred_element_type=jnp.float32) + bo_ref[...]
    xn = y * lax.rsqrt(jnp.mean(y * y, axis=-1, keepdims=True) + EPS)
    xn = xn * norm2_ref[...]
    h = jnp.dot(xn, w1_ref[...], preferred_element_type=jnp.float32) + b1_ref[...]
    h = 0.5 * h * (1.0 + lax.erf(h * np.float32(1.0 / np.sqrt(2.0))))
    o_ref[...] = y + jnp.dot(h, w2_ref[...], preferred_element_type=jnp.float32) + b2_ref[...]


def _tail(x, n0, n1, d0, d1, wo, bo, n2, w1, b1, w2, b2):
    row = pl.BlockSpec((BN, C), lambda i: (i, 0))
    drow = pl.BlockSpec((BN, 16), lambda i: (i, 0))
    vec = pl.BlockSpec((1, C), lambda i: (0, 0))
    expand = jnp.asarray(np.kron(np.eye(H), np.ones((1, DH))), dtype=jnp.float32)
    return pl.pallas_call(
        _tail_body,
        grid=(N // BN,),
        in_specs=[row, row, row, drow, drow,
                  pl.BlockSpec((H, C), lambda i: (0, 0)),
                  pl.BlockSpec((C, C), lambda i: (0, 0)), vec, vec,
                  pl.BlockSpec((C, FFN), lambda i: (0, 0)),
                  pl.BlockSpec((1, FFN), lambda i: (0, 0)),
                  pl.BlockSpec((FFN, C), lambda i: (0, 0)), vec],
        out_specs=row,
        out_shape=jax.ShapeDtypeStruct((N, C), jnp.float32),
    )(x, n0, n1, d0, d1, expand, wo, bo.reshape(1, C), n2.reshape(1, C),
      w1, b1.reshape(1, FFN), w2, b2.reshape(1, C))


def _in_proj_body(x_ref, w_ref, b_ref, o_ref):
    o_ref[...] = jnp.dot(x_ref[...], w_ref[...], preferred_element_type=jnp.float32) + b_ref[...]


def _in_proj(x, w, b):
    row = pl.BlockSpec((BN, C), lambda i: (i, 0))
    return pl.pallas_call(
        _in_proj_body,
        grid=(N // BN,),
        in_specs=[row, pl.BlockSpec((C, C), lambda i: (0, 0)),
                  pl.BlockSpec((1, C), lambda i: (0, 0))],
        out_specs=row,
        out_shape=jax.ShapeDtypeStruct((N, C), jnp.float32),
    )(x, w, b.reshape(1, C))


def _dyn_gather(vec, idx):
    return lax.gather(vec, idx[:, None],
                      dimension_numbers=lax.GatherDimensionNumbers(
                          offset_dims=(), collapsed_slice_dims=(0,),
                          start_index_map=(0,)),
                      slice_sizes=(1,),
                      mode=lax.GatherScatterMode.PROMISE_IN_BOUNDS)


def _edge_body(q_hbm, kv_hbm, rel_hbm, src_hbm, dst_hbm, et_hbm,
               zn_hbm, zd_hbm,
               onum, oden,
               qb0, qb1, kvb0, kvb1, wb0, wb1, xb0, xb1, relbuf,
               sb0, sb1, db0, db1, eb0, eb1, sdb0, sdb1,
               num_sh, den_sh, isem, gsem, ssem0, ssem1):
    qb = (qb0, qb1)
    kvb = (kvb0, kvb1)
    wb = (wb0, wb1)
    xb = (xb0, xb1)
    sb = (sb0, sb1)
    db = (db0, db1)
    eb = (eb0, eb1)
    sdb = (sdb0, sdb1)
    ssem = (ssem0, ssem1)

    cid = lax.axis_index("c")
    sid = lax.axis_index("s")
    wid = sid * 2 + cid
    pltpu.sync_copy(rel_hbm, relbuf)
    pltpu.sync_copy(zn_hbm, wb0)
    pltpu.sync_copy(zd_hbm, xb0)
    row0 = sid * NROW
    for t in range(NROW // CH):
        pltpu.sync_copy(wb0, num_sh.at[pl.ds(row0 + t * CH, CH)])
        pltpu.sync_copy(xb0, den_sh.at[pl.ds(row0 + t * CH, CH)])
    plsc.subcore_barrier()

    ebase = wid * EWP
    iota16 = lax.iota(jnp.int32, 16)

    c1 = pltpu.async_copy(src_hbm.at[pl.ds(ebase, CH)], sb0, isem)
    c2 = pltpu.async_copy(dst_hbm.at[pl.ds(ebase, CH)], db0, isem)
    c3 = pltpu.async_copy(et_hbm.at[pl.ds(ebase, CH)], eb0, isem)
    c1.wait()
    c2.wait()
    c3.wait()
    pltpu.async_copy(q_hbm.at[db0], qb0, gsem)
    pltpu.async_copy(kv_hbm.at[sb0], kvb0, gsem)
    pltpu.async_copy(src_hbm.at[pl.ds(ebase + CH, CH)], sb1, isem)
    pltpu.async_copy(dst_hbm.at[pl.ds(ebase + CH, CH)], db1, isem)
    pltpu.async_copy(et_hbm.at[pl.ds(ebase + CH, CH)], eb1, isem)

    def step(i2, carry):
        for b in (0, 1):
            ob = 1 - b
            cbase = ebase + (i2 * 2 + b) * CH
            pltpu.make_async_copy(q_hbm.at[pl.ds(0, CH)], qb[b], gsem).wait()
            pltpu.make_async_copy(kv_hbm.at[pl.ds(0, CH)], kvb[b], gsem).wait()
            pltpu.make_async_copy(src_hbm.at[pl.ds(0, CH)], sb[ob], isem).wait()
            pltpu.make_async_copy(dst_hbm.at[pl.ds(0, CH)], db[ob], isem).wait()
            pltpu.make_async_copy(et_hbm.at[pl.ds(0, CH)], eb[ob], isem).wait()
            pltpu.async_copy(q_hbm.at[db[ob]], qb[ob], gsem)
            pltpu.async_copy(kv_hbm.at[sb[ob]], kvb[ob], gsem)

            @pl.when(i2 >= 1)
            def _drain():
                pltpu.make_async_copy(zn_hbm, wb[b], ssem[b]).wait()
                pltpu.make_async_copy(zd_hbm, xb[b], ssem[b]).wait()

            def group(g, c2):
                et16 = eb[b][pl.ds(g * 16, 16)]
                for e0 in range(0, 16, 2):
                    EH = [(g * 16 + e0 + j, h) for j in (0, 1)
                          for h in range(H)]
                    ete = [_dyn_gather(et16, jnp.full((16,), e0 + j, jnp.int32))
                           for j in (0, 1)]
                    rbase = [ete[j] * C + iota16 for j in (0, 1)]
                    qv = [qb[b][er, pl.ds(h * DH, 16)] for er, h in EH]
                    kv = [kvb[b][er, pl.ds(h * DH, 16)] for er, h in EH]
                    rv = [plsc.load_gather(relbuf, [rbase[i // H] + h * DH])
                          for i, (er, h) in enumerate(EH)]
                    p = [qv[i] * (kv[i] + rv[i]) for i in range(16)]
                    for sh in (8, 4, 2, 1):
                        pg = [_dyn_gather(p[i], iota16 ^ sh) for i in range(16)]
                        p = [p[i] + pg[i] for i in range(16)]
                    ex = [jnp.exp(p[i] * np.float32(1.0 / np.sqrt(DH)))
                          for i in range(16)]
                    vv = [kvb[b][er, pl.ds(C + h * DH, 16)] for er, h in EH]
                    w = [(vv[i] + rv[i]) * ex[i] for i in range(16)]
                    for i, (er, h) in enumerate(EH):
                        wb[b][er, pl.ds(h * DH, 16)] = w[i]
                    for j in (0, 1):
                        exrow = jnp.zeros((16,), jnp.float32)
                        for h in range(H):
                            exrow = jnp.where(iota16 == h, ex[j * H + h], exrow)
                        xb[b][g * 16 + e0 + j, pl.ds(0, 16)] = exrow
                return c2
            lax.fori_loop(0, NG, group, 0)

            sdb[b][pl.ds(0, 16)] = db[b][pl.ds(0, 16)]
            sdb[b][pl.ds(16, 16)] = db[b][pl.ds(16, 16)]
            pltpu.async_copy(wb[b], num_sh.at[sdb[b]], ssem[b], add=True)
            pltpu.async_copy(xb[b], den_sh.at[sdb[b]], ssem[b], add=True)
            pltpu.async_copy(src_hbm.at[pl.ds(cbase + 2 * CH, CH)], sb[b], isem)
            pltpu.async_copy(dst_hbm.at[pl.ds(cbase + 2 * CH, CH)], db[b], isem)
            pltpu.async_copy(et_hbm.at[pl.ds(cbase + 2 * CH, CH)], eb[b], isem)
        return carry
    lax.fori_loop(0, NCHUNK // 2, step, 0)

    pltpu.make_async_copy(q_hbm.at[pl.ds(0, CH)], qb0, gsem).wait()
    pltpu.make_async_copy(kv_hbm.at[pl.ds(0, CH)], kvb0, gsem).wait()
    pltpu.make_async_copy(src_hbm.at[pl.ds(0, CH)], sb1, isem).wait()
    pltpu.make_async_copy(dst_hbm.at[pl.ds(0, CH)], db1, isem).wait()
    pltpu.make_async_copy(et_hbm.at[pl.ds(0, CH)], eb1, isem).wait()
    pltpu.make_async_copy(zn_hbm, wb0, ssem0).wait()
    pltpu.make_async_copy(zd_hbm, xb0, ssem0).wait()
    pltpu.make_async_copy(zn_hbm, wb1, ssem1).wait()
    pltpu.make_async_copy(zd_hbm, xb1, ssem1).wait()
    plsc.subcore_barrier()
    out_base = cid * NP + row0
    for t in range(NROW // CH):
        pltpu.sync_copy(num_sh.at[pl.ds(row0 + t * CH, CH)], wb0)
        pltpu.sync_copy(wb0, onum.at[pl.ds(out_base + t * CH, CH)])
        pltpu.sync_copy(den_sh.at[pl.ds(row0 + t * CH, CH)], xb0)
        pltpu.sync_copy(xb0, oden.at[pl.ds(out_base + t * CH, CH)])


def _edge_phase(q, kv, rel, srcp, dstp, etp):
    mesh = plsc.VectorSubcoreMesh(core_axis_name="c", subcore_axis_name="s")
    fn = pl.kernel(
        _edge_body,
        mesh=mesh,
        compiler_params=pltpu.CompilerParams(needs_layout_passes=False,
                                             use_tc_tiling_on_sc=False),
        out_type=[jax.ShapeDtypeStruct((2 * NP, C), jnp.float32),
                  jax.ShapeDtypeStruct((2 * NP, 16), jnp.float32)],
        scratch_types=[
            pltpu.VMEM((CH, C), jnp.float32),
            pltpu.VMEM((CH, C), jnp.float32),
            pltpu.VMEM((CH, 2 * C), jnp.float32),
            pltpu.VMEM((CH, 2 * C), jnp.float32),
            pltpu.VMEM((CH, C), jnp.float32),
            pltpu.VMEM((CH, C), jnp.float32),
            pltpu.VMEM((CH, 16), jnp.float32),
            pltpu.VMEM((CH, 16), jnp.float32),
            pltpu.VMEM((R * C,), jnp.float32),
            pltpu.VMEM((CH,), jnp.int32),
            pltpu.VMEM((CH,), jnp.int32),
            pltpu.VMEM((CH,), jnp.int32),
            pltpu.VMEM((CH,), jnp.int32),
            pltpu.VMEM((CH,), jnp.int32),
            pltpu.VMEM((CH,), jnp.int32),
            pltpu.VMEM((CH,), jnp.int32),
            pltpu.VMEM((CH,), jnp.int32),
            pltpu.VMEM_SHARED((NP, C), jnp.float32),
            pltpu.VMEM_SHARED((NP, 16), jnp.float32),
            pltpu.SemaphoreType.DMA,
            pltpu.SemaphoreType.DMA,
            pltpu.SemaphoreType.DMA,
            pltpu.SemaphoreType.DMA,
        ],
    )
    zn = jnp.zeros((CH, C), jnp.float32)
    zd = jnp.zeros((CH, 16), jnp.float32)
    onum, oden = fn(q, kv, rel.reshape(-1), srcp, dstp, etp, zn, zd)
    return onum, oden


def kernel(x, edge_index, edge_type, params):
    pad = EPAD - E
    srcp = jnp.concatenate([edge_index[0], jnp.zeros((pad,), edge_index.dtype)])
    dstp = jnp.concatenate([edge_index[1], jnp.full((pad,), N, edge_index.dtype)])
    etp = jnp.concatenate([edge_type, jnp.zeros((pad,), edge_type.dtype)])
    xpad = jnp.zeros((NP - N, C), jnp.float32)
    p0 = params["input_proj"]
    x = _in_proj(x, p0["w"], p0["b"])
    for p in params["blocks"]:
        xp = jnp.concatenate([x, xpad])
        q, kv = _qkv(xp, p["norm1"], p["q"]["w"], p["q"]["b"],
                     p["k"]["w"], p["k"]["b"], p["v"]["w"], p["v"]["b"])
        onum, oden = _edge_phase(q, kv, p["rel"], srcp, dstp, etp)
        x = _tail(x, onum[:N], onum[NP:NP + N], oden[:N], oden[NP:NP + N],
                  p["out"]["w"], p["out"]["b"], p["norm2"],
                  p["ffn1"]["w"], p["ffn1"]["b"], p["ffn2"]["w"], p["ffn2"]["b"])
    return x
